# Initial kernel scaffold; baseline (speedup 1.0000x reference)
import jax
import jax.numpy as jnp
from jax.experimental import pallas as pl


def kernel(input_ids, token_type_ids, tok_table, pos_table, type_table, gamma, beta):
    raise NotImplementedError("write your pallas kernel here")


# speedup vs baseline: 2.1511x; 2.1511x over previous
import functools

import jax
import jax.numpy as jnp
from jax import lax
from jax.experimental import pallas as pl
from jax.experimental.pallas import tpu as pltpu
from jax.experimental.pallas import tpu_sc as plsc

NC = 2
NS = 16
NW = NC * NS
CHUNK = 64


def _sc_gather(table, idx_flat, n_rows, hidden):
    per_w = n_rows // NW
    mesh = plsc.VectorSubcoreMesh(core_axis_name="c", subcore_axis_name="s")

    @functools.partial(
        pl.kernel,
        mesh=mesh,
        out_type=jax.ShapeDtypeStruct((n_rows, hidden), jnp.float32),
        scratch_types=[
            pltpu.VMEM((CHUNK,), jnp.int32),
            pltpu.VMEM((CHUNK, hidden), jnp.float32),
            pltpu.SemaphoreType.DMA,
        ],
    )
    def gather_kernel(table_hbm, idx_hbm, out_hbm, idx_v, rows_v, sem):
        wid = lax.axis_index("s") * NC + lax.axis_index("c")
        base = wid * per_w

        @pl.loop(0, per_w, step=CHUNK)
        def _(off):
            pltpu.sync_copy(idx_hbm.at[pl.ds(base + off, CHUNK)], idx_v)
            pltpu.async_copy(table_hbm.at[idx_v], rows_v, sem).wait()
            pltpu.sync_copy(rows_v, out_hbm.at[pl.ds(base + off, CHUNK)])

    return gather_kernel(table, idx_flat)


def _tc_body(g_ref, tt_ref, pos_ref, type_ref, gamma_ref, beta_ref, o_ref):
    x = g_ref[...]
    tt = tt_ref[...][..., None]
    t0 = type_ref[0]
    t1 = type_ref[1]
    t2 = type_ref[2]
    type_emb = jnp.where(tt == 0, t0, jnp.where(tt == 1, t1, t2))
    x = x + pos_ref[...][None] + type_emb
    mean = jnp.mean(x, axis=-1, keepdims=True)
    xc = x - mean
    var = jnp.mean(xc * xc, axis=-1, keepdims=True)
    inv = lax.rsqrt(var + 1e-12)
    o_ref[...] = xc * inv * gamma_ref[...] + beta_ref[...]


def _tc_add_ln(gathered, token_type_ids, pos_table, type_pad, gamma, beta):
    B, S = token_type_ids.shape
    H = gathered.shape[-1]
    SB = 8
    grid = (B // SB,)
    return pl.pallas_call(
        _tc_body,
        grid=grid,
        in_specs=[
            pl.BlockSpec((SB, S, H), lambda i: (i, 0, 0)),
            pl.BlockSpec((SB, S), lambda i: (i, 0)),
            pl.BlockSpec((S, H), lambda i: (0, 0)),
            pl.BlockSpec((8, H), lambda i: (0, 0)),
            pl.BlockSpec((H,), lambda i: (0,)),
            pl.BlockSpec((H,), lambda i: (0,)),
        ],
        out_specs=pl.BlockSpec((SB, S, H), lambda i: (i, 0, 0)),
        out_shape=jax.ShapeDtypeStruct((B, S, H), jnp.float32),
        compiler_params=pltpu.CompilerParams(
            dimension_semantics=("parallel",),
        ),
    )(gathered, token_type_ids, pos_table, type_pad, gamma, beta)


def kernel(input_ids, token_type_ids, tok_table, pos_table, type_table, gamma, beta):
    B, S = input_ids.shape
    V, H = tok_table.shape
    n = B * S
    ids_flat = input_ids.reshape(n).astype(jnp.int32)
    type_pad = jnp.zeros((8, H), jnp.float32).at[:3].set(type_table)

    gathered = _sc_gather(tok_table, ids_flat, n, H)
    gathered = gathered.reshape(B, S, H)
    return _tc_add_ln(gathered, token_type_ids, pos_table, type_pad, gamma, beta)
